# baseline (speedup 1.0000x reference)
import functools

import jax
import jax.numpy as jnp
from jax import lax
from jax.experimental import pallas as pl
from jax.experimental.pallas import tpu as pltpu
from jax.experimental.pallas import tpu_sc as plsc

B, C, H, W = 8, 256, 56, 56
PH, PW = 7, 7
S = PH * PW
NPAD = 1024
NC, NS = 2, 16
NW = NC * NS
RPW = NPAD // NW
SCALE = 0.25
OFFSET = 0.5

_mesh = plsc.VectorSubcoreMesh(
    core_axis_name="c", subcore_axis_name="s", num_cores=NC, num_subcores=NS
)


_OUT_TYPE = jax.ShapeDtypeStruct((NPAD, C, S), jnp.float32)
_SCRATCH = [
    pltpu.VMEM((RPW * 8 + 16,), jnp.float32),
    pltpu.VMEM((64,), jnp.int32),
    pltpu.VMEM((64,), jnp.int32),
    pltpu.VMEM((64,), jnp.int32),
    pltpu.VMEM((64,), jnp.int32),
    pltpu.VMEM((4 * 64 + 16,), jnp.float32),
    pltpu.VMEM((64, C), jnp.float32),
    pltpu.VMEM((64, C), jnp.float32),
    pltpu.VMEM((64, C), jnp.float32),
    pltpu.VMEM((64, C), jnp.float32),
    pltpu.VMEM((C, S), jnp.float32),
    pltpu.SemaphoreType.DMA,
]


def _sc_body(fm_hbm, rois_hbm, out_hbm, rois_v,
             idx0, idx1, idx2, idx3, w_v, rows0, rows1, rows2, rows3,
             out_t, sem):
    idxs = (idx0, idx1, idx2, idx3)
    rows = (rows0, rows1, rows2, rows3)
    wid = lax.axis_index("s") * NC + lax.axis_index("c")
    base = wid * RPW
    pltpu.sync_copy(rois_hbm.at[pl.ds(base * 8, RPW * 8)], rois_v.at[pl.ds(0, RPW * 8)])
    ci = lax.iota(jnp.int32, 16)

    zi = jnp.zeros((16,), jnp.int32)

    def splat(ref, i):
        return plsc.load_gather(ref, [zi + i])

    def roi_body(r, carry):
        off = r * 8
        bi = splat(rois_v, off + 0).astype(jnp.int32)
        x1 = splat(rois_v, off + 1) * SCALE - OFFSET
        y1 = splat(rois_v, off + 2) * SCALE - OFFSET
        x2 = splat(rois_v, off + 3) * SCALE - OFFSET
        y2 = splat(rois_v, off + 4) * SCALE - OFFSET
        bin_h = (y2 - y1) / PH
        bin_w = (x2 - x1) / PW
        rowbase = bi * (H * W)
        for k in range(4):
            ii = ci + k * 16
            ph = ii // PW
            pw = ii % PW
            yy = y1 + (ph.astype(jnp.float32) + 0.5) * bin_h
            xx = x1 + (pw.astype(jnp.float32) + 0.5) * bin_w
            valid = (yy > -1.0) & (yy < float(H)) & (xx > -1.0) & (xx < float(W))
            y = jnp.clip(yy, 0.0, float(H - 1))
            x = jnp.clip(xx, 0.0, float(W - 1))
            y0 = y.astype(jnp.int32)
            x0 = x.astype(jnp.int32)
            y1i = jnp.minimum(y0 + 1, H - 1)
            x1i = jnp.minimum(x0 + 1, W - 1)
            ly = y - y0.astype(jnp.float32)
            lx = x - x0.astype(jnp.float32)
            hy = 1.0 - ly
            hx = 1.0 - lx
            vm = jnp.where(valid, 1.0, 0.0).astype(jnp.float32)
            sl = pl.ds(k * 16, 16)
            idx0[sl] = rowbase + y0 * W + x0
            idx1[sl] = rowbase + y0 * W + x1i
            idx2[sl] = rowbase + y1i * W + x0
            idx3[sl] = rowbase + y1i * W + x1i
            wi = (ci + k * 16) * 4
            plsc.store_scatter(w_v, [wi + 0], hy * hx * vm)
            plsc.store_scatter(w_v, [wi + 1], hy * lx * vm)
            plsc.store_scatter(w_v, [wi + 2], ly * hx * vm)
            plsc.store_scatter(w_v, [wi + 3], ly * lx * vm)
        cps = [
            pltpu.async_copy(fm_hbm.at[idxs[j]], rows[j], sem)
            for j in range(4)
        ]
        for cp in cps:
            cp.wait()

        def s_body(s, c2):
            w00 = splat(w_v, s * 4 + 0)
            w01 = splat(w_v, s * 4 + 1)
            w10 = splat(w_v, s * 4 + 2)
            w11 = splat(w_v, s * 4 + 3)
            si = zi + s
            for k in range(C // 16):
                sl = pl.ds(k * 16, 16)
                acc = (
                    w00 * rows0[s, sl]
                    + w01 * rows1[s, sl]
                    + w10 * rows2[s, sl]
                    + w11 * rows3[s, sl]
                )
                plsc.store_scatter(out_t, [ci + k * 16, si], acc)
            return c2

        lax.fori_loop(0, S, s_body, 0)
        pltpu.sync_copy(out_t, out_hbm.at[base + r])
        return carry

    lax.fori_loop(0, RPW, roi_body, 0)


_roi_align_sc = pl.kernel(
    _sc_body,
    out_type=_OUT_TYPE,
    mesh=_mesh,
    scratch_types=_SCRATCH,
    compiler_params=pltpu.CompilerParams(needs_layout_passes=False),
)


def kernel(featuremaps, rois):
    fm_rows = jnp.transpose(featuremaps, (0, 2, 3, 1)).reshape(B * H * W, C)
    n = rois.shape[0]
    rois_p = jnp.zeros((NPAD, 8), jnp.float32).at[:n, :5].set(rois).reshape(-1)
    out = _roi_align_sc(fm_rows, rois_p)
    return out[:n].reshape(n, C, PH, PW)


# speedup vs baseline: 1.4759x; 1.4759x over previous
import jax
import jax.numpy as jnp
from jax import lax
from jax.experimental import pallas as pl
from jax.experimental.pallas import tpu as pltpu
from jax.experimental.pallas import tpu_sc as plsc

B, C, H, W = 8, 256, 56, 56
PH, PW = 7, 7
S = PH * PW
NPAD = 1024
NC, NS = 2, 16
NW = NC * NS
RPW = NPAD // NW
SCALE = 0.25
OFFSET = 0.5

_mesh = plsc.VectorSubcoreMesh(
    core_axis_name="c", subcore_axis_name="s", num_cores=NC, num_subcores=NS
)

_OUT_TYPE = jax.ShapeDtypeStruct((NPAD, C * S), jnp.float32)


def _idx_bufs():
    return [pltpu.VMEM((S,), jnp.int32) for _ in range(4)]


def _row_bufs():
    return [pltpu.VMEM((S, C), jnp.float32) for _ in range(4)]


_SCRATCH = (
    [pltpu.VMEM((RPW * 8 + 16,), jnp.float32)]
    + _idx_bufs() + _idx_bufs()
    + [pltpu.VMEM((4 * S + 16,), jnp.float32)] * 2
    + _row_bufs() + _row_bufs()
    + [
        pltpu.VMEM((C * S,), jnp.float32),
        pltpu.SemaphoreType.DMA,
        pltpu.SemaphoreType.DMA,
    ]
)


def _sc_body(fm_hbm, rois_hbm, out_hbm, rois_v,
             ia0, ia1, ia2, ia3, ib0, ib1, ib2, ib3, wa, wb,
             ra0, ra1, ra2, ra3, rb0, rb1, rb2, rb3,
             out_t, sema, semb):
    bufs = (
        ((ia0, ia1, ia2, ia3), wa, (ra0, ra1, ra2, ra3), sema),
        ((ib0, ib1, ib2, ib3), wb, (rb0, rb1, rb2, rb3), semb),
    )
    wid = lax.axis_index("s") * NC + lax.axis_index("c")
    base = wid * RPW
    pltpu.sync_copy(
        rois_hbm.at[pl.ds(base * 8, RPW * 8)], rois_v.at[pl.ds(0, RPW * 8)]
    )
    ci = lax.iota(jnp.int32, 16)
    zi = jnp.zeros((16,), jnp.int32)

    def splat(ref, i):
        return plsc.load_gather(ref, [zi + i])

    def compute(r, bi_):
        idxs, w_v, _, _ = bufs[bi_]
        off = r * 8
        b = splat(rois_v, off + 0).astype(jnp.int32)
        x1 = splat(rois_v, off + 1) * SCALE - OFFSET
        y1 = splat(rois_v, off + 2) * SCALE - OFFSET
        x2 = splat(rois_v, off + 3) * SCALE - OFFSET
        y2 = splat(rois_v, off + 4) * SCALE - OFFSET
        bin_h = (y2 - y1) / PH
        bin_w = (x2 - x1) / PW
        rowbase = b * (H * W)
        for kb in (0, 16, 32, 33):
            ii = ci + kb
            ph = ii // PW
            pw = ii % PW
            yy = y1 + (ph.astype(jnp.float32) + 0.5) * bin_h
            xx = x1 + (pw.astype(jnp.float32) + 0.5) * bin_w
            valid = (yy > -1.0) & (yy < float(H)) & (xx > -1.0) & (xx < float(W))
            y = jnp.clip(yy, 0.0, float(H - 1))
            x = jnp.clip(xx, 0.0, float(W - 1))
            y0 = y.astype(jnp.int32)
            x0 = x.astype(jnp.int32)
            y1i = jnp.minimum(y0 + 1, H - 1)
            x1i = jnp.minimum(x0 + 1, W - 1)
            ly = y - y0.astype(jnp.float32)
            lx = x - x0.astype(jnp.float32)
            hy = 1.0 - ly
            hx = 1.0 - lx
            vm = jnp.where(valid, 1.0, 0.0).astype(jnp.float32)
            rb_ = rowbase + y0 * W
            rt_ = rowbase + y1i * W
            sl = pl.ds(kb, 16)
            idxs[0][sl] = rb_ + x0
            idxs[1][sl] = rb_ + x1i
            idxs[2][sl] = rt_ + x0
            idxs[3][sl] = rt_ + x1i
            wi = ii * 4
            plsc.store_scatter(w_v, [wi + 0], hy * hx * vm)
            plsc.store_scatter(w_v, [wi + 1], hy * lx * vm)
            plsc.store_scatter(w_v, [wi + 2], ly * hx * vm)
            plsc.store_scatter(w_v, [wi + 3], ly * lx * vm)

    def fire(bi_):
        idxs, _, rows, sem = bufs[bi_]
        return [
            pltpu.async_copy(fm_hbm.at[idxs[j]], rows[j], sem) for j in range(4)
        ]

    def combine_out(r, bi_):
        _, w_v, rows, _ = bufs[bi_]

        def s_body(s, c2):
            w00 = splat(w_v, s * 4 + 0)
            w01 = splat(w_v, s * 4 + 1)
            w10 = splat(w_v, s * 4 + 2)
            w11 = splat(w_v, s * 4 + 3)
            si = zi + s
            for k in range(C // 16):
                sl = pl.ds(k * 16, 16)
                acc = (
                    w00 * rows[0][s, sl]
                    + w01 * rows[1][s, sl]
                    + w10 * rows[2][s, sl]
                    + w11 * rows[3][s, sl]
                )
                plsc.store_scatter(out_t, [(ci + k * 16) * S + si], acc)
            return c2

        lax.fori_loop(0, S, s_body, 0)
        pltpu.sync_copy(out_t, out_hbm.at[base + r])

    def pair_body(t, carry):
        r = t * 2
        compute(r, 0)
        cps_a = fire(0)
        compute(r + 1, 1)
        cps_b = fire(1)
        for cp in cps_a:
            cp.wait()
        combine_out(r, 0)
        for cp in cps_b:
            cp.wait()
        combine_out(r + 1, 1)
        return carry

    lax.fori_loop(0, RPW // 2, pair_body, 0)


_roi_align_sc = pl.kernel(
    _sc_body,
    out_type=_OUT_TYPE,
    mesh=_mesh,
    scratch_types=_SCRATCH,
    compiler_params=pltpu.CompilerParams(needs_layout_passes=False),
)


def kernel(featuremaps, rois):
    fm_rows = jnp.transpose(featuremaps, (0, 2, 3, 1)).reshape(B * H * W, C)
    n = rois.shape[0]
    rois_p = jnp.zeros((NPAD, 8), jnp.float32).at[:n, :5].set(rois).reshape(-1)
    out = _roi_align_sc(fm_rows, rois_p)
    return out[:n].reshape(n, C, PH, PW)
